# baseline (speedup 1.0000x reference)
import jax
import jax.numpy as jnp
from jax import lax
from jax.experimental import pallas as pl
from jax.experimental.pallas import tpu as pltpu
from jax.experimental.pallas import tpu_sc as plsc

N = 10000
E = 320000
D_IN = 128
HID = 256
N_PAD = 10240
BLK = 1024

NC = 2
NS = 16
NW = NC * NS
EPW = E // NW
EB = 128
PADW = 10240
NB = PADW // EB
EB2 = 80
NB2 = PADW // EB2
BPS = 8
SB = NB2 // BPS
RPT = N_PAD // NS


def _mesh():
    return plsc.VectorSubcoreMesh(core_axis_name="c", subcore_axis_name="s")


_Z16 = lambda: jnp.zeros((16,), jnp.float32)

_GDN = lax.GatherDimensionNumbers(offset_dims=(), collapsed_slice_dims=(0,),
                                  start_index_map=(0,))


def _splat(w16, j):
    idx = jnp.full((16, 1), j, jnp.int32)
    return lax.gather(w16, idx, _GDN, slice_sizes=(1,),
                      mode=lax.GatherScatterMode.PROMISE_IN_BOUNDS)


def _dense_body(x_ref, wlinT_ref, blin_ref, wc0T_ref, wmlp_ref, bmlp_ref,
                dis_ref, g1_ref, mlp_ref):
    xl = jnp.dot(x_ref[...], wlinT_ref[...],
                 preferred_element_type=jnp.float32) + blin_ref[...]
    g1_ref[...] = dis_ref[...] * jnp.dot(xl, wc0T_ref[...],
                                         preferred_element_type=jnp.float32)
    mlp_ref[...] = jnp.sum(xl * wmlp_ref[...], axis=1, keepdims=True) \
        + bmlp_ref[...]


def _dense(x_pad, W_linT, b_lin2, W_c0T, w_mlp2, b_mlp2, dis2):
    return pl.pallas_call(
        _dense_body,
        grid=(N_PAD // BLK,),
        in_specs=[
            pl.BlockSpec((BLK, D_IN), lambda i: (i, 0)),
            pl.BlockSpec((D_IN, HID), lambda i: (0, 0)),
            pl.BlockSpec((1, HID), lambda i: (0, 0)),
            pl.BlockSpec((HID, D_IN), lambda i: (0, 0)),
            pl.BlockSpec((1, HID), lambda i: (0, 0)),
            pl.BlockSpec((1, 1), lambda i: (0, 0)),
            pl.BlockSpec((BLK, 1), lambda i: (i, 0)),
        ],
        out_specs=[
            pl.BlockSpec((BLK, D_IN), lambda i: (i, 0)),
            pl.BlockSpec((BLK, 1), lambda i: (i, 0)),
        ],
        out_shape=[
            jax.ShapeDtypeStruct((N_PAD, D_IN), jnp.float32),
            jax.ShapeDtypeStruct((N_PAD, 1), jnp.float32),
        ],
    )(x_pad, W_linT, b_lin2, W_c0T, w_mlp2, b_mlp2, dis2)


def _ew_body(xr_ref, m_ref, out_ref):
    out_ref[...] = jnp.dot(xr_ref[...], m_ref[...],
                           preferred_element_type=jnp.float32)


def _edge_mean(ef_rows, m):
    rows = ef_rows.shape[0]
    blk = rows // 8
    return pl.pallas_call(
        _ew_body,
        grid=(8,),
        in_specs=[
            pl.BlockSpec((blk, 128), lambda i: (i, 0)),
            pl.BlockSpec((128, 8), lambda i: (0, 0)),
        ],
        out_specs=pl.BlockSpec((blk, 8), lambda i: (i, 0)),
        out_shape=jax.ShapeDtypeStruct((rows, 8), jnp.float32),
    )(ef_rows, m)


def _post1_body(accp_ref, g1_ref, dis_ref, bc0_ref, wc1_ref, g2_ref):
    acc = accp_ref[0] + accp_ref[1]
    disb = dis_ref[...]
    xc1 = jnp.maximum(disb * (acc + g1_ref[...]) + bc0_ref[...], 0.0)
    h2 = jnp.sum(xc1 * wc1_ref[...], axis=1, keepdims=True)
    g2_ref[...] = disb * h2


def _post1(accp, h1, dis2, bc0, wc1):
    return pl.pallas_call(
        _post1_body,
        grid=(N_PAD // BLK,),
        in_specs=[
            pl.BlockSpec((NC, BLK, D_IN), lambda i: (0, i, 0)),
            pl.BlockSpec((BLK, D_IN), lambda i: (i, 0)),
            pl.BlockSpec((BLK, 1), lambda i: (i, 0)),
            pl.BlockSpec((1, D_IN), lambda i: (0, 0)),
            pl.BlockSpec((1, D_IN), lambda i: (0, 0)),
        ],
        out_specs=pl.BlockSpec((BLK, 1), lambda i: (i, 0)),
        out_shape=jax.ShapeDtypeStruct((N_PAD, 1), jnp.float32),
    )(accp, h1, dis2, bc0, wc1)


def _final_body(acc2p_ref, g2_ref, dis_ref, mlp_ref, bc1_ref, out_ref):
    a = acc2p_ref[0] + acc2p_ref[1]
    disb = dis_ref[...]
    xc2 = jnp.maximum(disb * (a + g2_ref[...]) + bc1_ref[...], 0.0)
    out_ref[...] = xc2 + mlp_ref[...]


def _final(acc2p3, g2, dis2, mlp, bc1):
    return pl.pallas_call(
        _final_body,
        grid=(N_PAD // BLK,),
        in_specs=[
            pl.BlockSpec((NC, BLK, 1), lambda i: (0, i, 0)),
            pl.BlockSpec((BLK, 1), lambda i: (i, 0)),
            pl.BlockSpec((BLK, 1), lambda i: (i, 0)),
            pl.BlockSpec((BLK, 1), lambda i: (i, 0)),
            pl.BlockSpec((1, 1), lambda i: (0, 0)),
        ],
        out_specs=pl.BlockSpec((BLK, 1), lambda i: (i, 0)),
        out_shape=jax.ShapeDtypeStruct((N_PAD, 1), jnp.float32),
    )(acc2p3, g2, dis2, mlp, bc1)


def _sc_deg_body(dst_hbm, ew_hbm, out_hbm, acc_sh, zbuf, dstv, ewv, sems):
    c = lax.axis_index("c")
    s = lax.axis_index("s")
    wid = c * NS + s

    def zero_body(i, carry):
        zbuf[pl.ds(i * 16, 16)] = _Z16()
        return carry

    lax.fori_loop(0, RPT // 16, zero_body, None)
    pltpu.sync_copy(zbuf, acc_sh.at[pl.ds(s * RPT, RPT)])
    plsc.subcore_barrier()

    pltpu.sync_copy(dst_hbm.at[wid], dstv)
    pltpu.sync_copy(ew_hbm.at[wid], ewv)

    def fire_body(b, carry):
        pltpu.async_copy(ewv.at[b], acc_sh.at[dstv.at[b]], sems, add=True)
        return carry

    lax.fori_loop(0, NB, fire_body, None)

    def drain_body(b, carry):
        pltpu.make_async_copy(ewv.at[b], acc_sh.at[dstv.at[b]], sems).wait()
        return carry

    lax.fori_loop(0, NB, drain_body, None)
    plsc.subcore_barrier()
    pltpu.sync_copy(acc_sh.at[pl.ds(s * RPT, RPT)],
                    out_hbm.at[c, pl.ds(s * RPT, RPT)])


def _sc_deg(dst3, ew3):
    f = pl.kernel(
        _sc_deg_body,
        out_type=jax.ShapeDtypeStruct((NC, N_PAD), jnp.float32),
        mesh=_mesh(),
        scratch_types=[
            pltpu.VMEM_SHARED((N_PAD,), jnp.float32),
            pltpu.VMEM((RPT,), jnp.float32),
            pltpu.VMEM((NB, EB), jnp.int32),
            pltpu.VMEM((NB, EB), jnp.float32),
            pltpu.SemaphoreType.DMA,
        ],
    )
    return f(dst3, ew3)


def _scale_rows(buf, ewv, b):
    def group_body(gi, carry):
        sl = pl.ds(gi * 16, 16)
        w16 = ewv[b, sl]
        for j in range(16):
            wj = _splat(w16, j)
            e = gi * 16 + j
            for q in range(D_IN // 16):
                qs = pl.ds(q * 16, 16)
                buf[e, qs] = buf[e, qs] * wj
        return carry

    lax.fori_loop(0, EB2 // 16, group_body, None)


def _sc_l1_body(g1_hbm, src_hbm, dst_hbm, ew_hbm, out_hbm,
                acc_sh, srcv, dstv, ewv, buf0, buf1, sem0, sem1):
    c = lax.axis_index("c")
    s = lax.axis_index("s")
    wid = c * NS + s

    def zero_body(i, carry):
        for q in range(D_IN // 16):
            buf0[i, pl.ds(q * 16, 16)] = _Z16()
        return carry

    lax.fori_loop(0, EB2, zero_body, None)
    for kk in range(RPT // EB2):
        pltpu.sync_copy(buf0, acc_sh.at[pl.ds(s * RPT + kk * EB2, EB2)])
    plsc.subcore_barrier()

    dummy = g1_hbm.at[pl.ds(0, EB2)]

    def super_body(sb, carry):
        pltpu.sync_copy(src_hbm.at[wid, sb], srcv)
        pltpu.sync_copy(dst_hbm.at[wid, sb], dstv)
        pltpu.sync_copy(ew_hbm.at[wid, sb], ewv)
        pltpu.async_copy(g1_hbm.at[srcv.at[0]], buf0, sem0)

        def pair_body(k, carry1):
            b0 = 2 * k
            b1 = b0 + 1
            pltpu.async_copy(g1_hbm.at[srcv.at[b1]], buf1, sem1)
            pltpu.make_async_copy(dummy, buf0, sem0).wait()
            _scale_rows(buf0, ewv, b0)
            pltpu.sync_copy(buf0, acc_sh.at[dstv.at[b0]], add=True)

            @pl.when(k < BPS // 2 - 1)
            def _():
                pltpu.async_copy(g1_hbm.at[srcv.at[b0 + 2]], buf0, sem0)

            pltpu.make_async_copy(dummy, buf1, sem1).wait()
            _scale_rows(buf1, ewv, b1)
            pltpu.sync_copy(buf1, acc_sh.at[dstv.at[b1]], add=True)
            return carry1

        lax.fori_loop(0, BPS // 2, pair_body, None)
        return carry

    lax.fori_loop(0, SB, super_body, None)
    plsc.subcore_barrier()
    pltpu.sync_copy(acc_sh.at[pl.ds(s * RPT, RPT)],
                    out_hbm.at[c, pl.ds(s * RPT, RPT)])


def _sc_l1(g1, src4, dst4, ew4):
    f = pl.kernel(
        _sc_l1_body,
        out_type=jax.ShapeDtypeStruct((NC, N_PAD, D_IN), jnp.float32),
        mesh=_mesh(),
        scratch_types=[
            pltpu.VMEM_SHARED((N_PAD, D_IN), jnp.float32),
            pltpu.VMEM((BPS, EB2), jnp.int32),
            pltpu.VMEM((BPS, EB2), jnp.int32),
            pltpu.VMEM((BPS, EB2), jnp.float32),
            pltpu.VMEM((EB2, D_IN), jnp.float32),
            pltpu.VMEM((EB2, D_IN), jnp.float32),
            pltpu.SemaphoreType.DMA,
            pltpu.SemaphoreType.DMA,
        ],
    )
    return f(g1, src4, dst4, ew4)


def _sc_l2_body(g2_hbm, src_hbm, dst_hbm, ew_hbm, out_hbm,
                acc_sh, zbuf, srcv, dstv, ewv, gvals, valv, semg, sems):
    c = lax.axis_index("c")
    s = lax.axis_index("s")
    wid = c * NS + s

    def zero_body(i, carry):
        zbuf[pl.ds(i * 16, 16)] = _Z16()
        return carry

    lax.fori_loop(0, RPT // 16, zero_body, None)
    pltpu.sync_copy(zbuf, acc_sh.at[pl.ds(s * RPT, RPT)])
    plsc.subcore_barrier()

    pltpu.sync_copy(src_hbm.at[wid], srcv)
    pltpu.sync_copy(dst_hbm.at[wid], dstv)
    pltpu.sync_copy(ew_hbm.at[wid], ewv)

    def fire_g(b, carry):
        pltpu.async_copy(g2_hbm.at[srcv.at[b]], gvals.at[b], semg)
        return carry

    lax.fori_loop(0, NB, fire_g, None)

    def drain_g(b, carry):
        pltpu.make_async_copy(g2_hbm.at[srcv.at[b]], gvals.at[b], semg).wait()
        return carry

    lax.fori_loop(0, NB, drain_g, None)

    def scale_b(b, carry):
        for q in range(EB // 16):
            sl = pl.ds(q * 16, 16)
            valv[b, sl] = gvals[b, sl] * ewv[b, sl]
        return carry

    lax.fori_loop(0, NB, scale_b, None)

    def fire_s(b, carry):
        pltpu.async_copy(valv.at[b], acc_sh.at[dstv.at[b]], sems, add=True)
        return carry

    lax.fori_loop(0, NB, fire_s, None)

    def drain_s(b, carry):
        pltpu.make_async_copy(valv.at[b], acc_sh.at[dstv.at[b]], sems).wait()
        return carry

    lax.fori_loop(0, NB, drain_s, None)
    plsc.subcore_barrier()
    pltpu.sync_copy(acc_sh.at[pl.ds(s * RPT, RPT)],
                    out_hbm.at[c, pl.ds(s * RPT, RPT)])


def _sc_l2(g2, src3, dst3, ew3):
    f = pl.kernel(
        _sc_l2_body,
        out_type=jax.ShapeDtypeStruct((NC, N_PAD), jnp.float32),
        mesh=_mesh(),
        scratch_types=[
            pltpu.VMEM_SHARED((N_PAD,), jnp.float32),
            pltpu.VMEM((RPT,), jnp.float32),
            pltpu.VMEM((NB, EB), jnp.int32),
            pltpu.VMEM((NB, EB), jnp.int32),
            pltpu.VMEM((NB, EB), jnp.float32),
            pltpu.VMEM((NB, EB), jnp.float32),
            pltpu.VMEM((NB, EB), jnp.float32),
            pltpu.SemaphoreType.DMA,
            pltpu.SemaphoreType.DMA,
        ],
    )
    return f(g2, src3, dst3, ew3)


def kernel(x, edge_index, edge_feature, W_lin, b_lin, W_c0, b_c0, W_c1, b_c1,
           W_m0, b_m0, W_m1, b_m1):
    src_w = edge_index[0].reshape(NW, EPW)
    dst_w = edge_index[1].reshape(NW, EPW)

    m = jnp.kron(jnp.eye(8, dtype=jnp.float32),
                 jnp.full((16, 1), 1.0 / 16, jnp.float32))
    ew_w = _edge_mean(edge_feature.reshape(E // 8, 128), m).reshape(NW, EPW)

    def _padded(a, fill):
        fills = jnp.full((NW, PADW - EPW), fill, a.dtype)
        return jnp.concatenate([a, fills], axis=1)

    src_p = _padded(src_w, N)
    dst_p = _padded(dst_w, N)
    ew_p = _padded(ew_w, 0)
    src3 = src_p.reshape(NW, NB, EB)
    dst3 = dst_p.reshape(NW, NB, EB)
    ew3 = ew_p.reshape(NW, NB, EB)
    src4 = src_p.reshape(NW, SB, BPS, EB2)
    dst4 = dst_p.reshape(NW, SB, BPS, EB2)
    ew4 = ew_p.reshape(NW, SB, BPS, EB2)

    w_mlp = (W_m1 @ W_m0)[0]
    b_mlp = (b_m1 + W_m1 @ b_m0)[0]

    degp = _sc_deg(dst3, ew3)
    dis = lax.rsqrt(degp[0] + degp[1] + 1.0)
    dis2 = dis[:, None]

    x_pad = jnp.pad(x, ((0, N_PAD - N), (0, 0)))
    g1, mlp = _dense(x_pad, W_lin.T, b_lin[None, :], W_c0.T,
                     w_mlp[None, :], b_mlp[None, None], dis2)

    acc1p = _sc_l1(g1, src4, dst4, ew4)
    g2 = _post1(acc1p, g1, dis2, b_c0[None, :], W_c1)

    acc2p = _sc_l2(g2.reshape(N_PAD), src3, dst3, ew3)

    out = _final(acc2p.reshape(NC, N_PAD, 1), g2, dis2, mlp,
                 b_c1[None, :])
    return out[:N, 0]


# speedup vs baseline: 1.3027x; 1.3027x over previous
import jax
import jax.numpy as jnp
from jax import lax
from jax.experimental import pallas as pl
from jax.experimental.pallas import tpu as pltpu
from jax.experimental.pallas import tpu_sc as plsc

N = 10000
E = 320000
D_IN = 128
HID = 256
N_PAD = 10240
BLK = 1024

NC = 2
NS = 16
NW = NC * NS
EPW = E // NW
EB = 128
PADW = 10240
NB = PADW // EB
EB2 = 80
PADW2 = 10080
NB2 = PADW2 // EB2
BPS = 6
SB = NB2 // BPS
RPT = N_PAD // NS


def _mesh():
    return plsc.VectorSubcoreMesh(core_axis_name="c", subcore_axis_name="s")


_Z16 = lambda: jnp.zeros((16,), jnp.float32)

_GDN = lax.GatherDimensionNumbers(offset_dims=(), collapsed_slice_dims=(0,),
                                  start_index_map=(0,))


def _splat(w16, j):
    idx = jnp.full((16, 1), j, jnp.int32)
    return lax.gather(w16, idx, _GDN, slice_sizes=(1,),
                      mode=lax.GatherScatterMode.PROMISE_IN_BOUNDS)


def _dense_body(x_ref, wlinT_ref, blin_ref, wc0T_ref, wmlp_ref, bmlp_ref,
                dis_ref, g1_ref, mlp_ref):
    xl = jnp.dot(x_ref[...], wlinT_ref[...],
                 preferred_element_type=jnp.float32) + blin_ref[...]
    g1_ref[...] = dis_ref[...] * jnp.dot(xl, wc0T_ref[...],
                                         preferred_element_type=jnp.float32)
    mlp_ref[...] = jnp.sum(xl * wmlp_ref[...], axis=1, keepdims=True) \
        + bmlp_ref[...]


def _dense(x_pad, W_linT, b_lin2, W_c0T, w_mlp2, b_mlp2, dis2):
    return pl.pallas_call(
        _dense_body,
        grid=(N_PAD // BLK,),
        in_specs=[
            pl.BlockSpec((BLK, D_IN), lambda i: (i, 0)),
            pl.BlockSpec((D_IN, HID), lambda i: (0, 0)),
            pl.BlockSpec((1, HID), lambda i: (0, 0)),
            pl.BlockSpec((HID, D_IN), lambda i: (0, 0)),
            pl.BlockSpec((1, HID), lambda i: (0, 0)),
            pl.BlockSpec((1, 1), lambda i: (0, 0)),
            pl.BlockSpec((BLK, 1), lambda i: (i, 0)),
        ],
        out_specs=[
            pl.BlockSpec((BLK, D_IN), lambda i: (i, 0)),
            pl.BlockSpec((BLK, 1), lambda i: (i, 0)),
        ],
        out_shape=[
            jax.ShapeDtypeStruct((N_PAD, D_IN), jnp.float32),
            jax.ShapeDtypeStruct((N_PAD, 1), jnp.float32),
        ],
    )(x_pad, W_linT, b_lin2, W_c0T, w_mlp2, b_mlp2, dis2)


def _ew_body(xr_ref, m_ref, out_ref):
    out_ref[...] = jnp.dot(xr_ref[...], m_ref[...],
                           preferred_element_type=jnp.float32)


def _edge_mean(ef_rows, m):
    rows = ef_rows.shape[0]
    blk = rows // 8
    return pl.pallas_call(
        _ew_body,
        grid=(8,),
        in_specs=[
            pl.BlockSpec((blk, 128), lambda i: (i, 0)),
            pl.BlockSpec((128, 8), lambda i: (0, 0)),
        ],
        out_specs=pl.BlockSpec((blk, 8), lambda i: (i, 0)),
        out_shape=jax.ShapeDtypeStruct((rows, 8), jnp.float32),
    )(ef_rows, m)


def _post1_body(accp_ref, g1_ref, dis_ref, bc0_ref, wc1_ref, g2_ref):
    acc = accp_ref[0] + accp_ref[1]
    disb = dis_ref[...]
    xc1 = jnp.maximum(disb * (acc + g1_ref[...]) + bc0_ref[...], 0.0)
    h2 = jnp.sum(xc1 * wc1_ref[...], axis=1, keepdims=True)
    g2_ref[...] = disb * h2


def _post1(accp, h1, dis2, bc0, wc1):
    return pl.pallas_call(
        _post1_body,
        grid=(N_PAD // BLK,),
        in_specs=[
            pl.BlockSpec((NC, BLK, D_IN), lambda i: (0, i, 0)),
            pl.BlockSpec((BLK, D_IN), lambda i: (i, 0)),
            pl.BlockSpec((BLK, 1), lambda i: (i, 0)),
            pl.BlockSpec((1, D_IN), lambda i: (0, 0)),
            pl.BlockSpec((1, D_IN), lambda i: (0, 0)),
        ],
        out_specs=pl.BlockSpec((BLK, 1), lambda i: (i, 0)),
        out_shape=jax.ShapeDtypeStruct((N_PAD, 1), jnp.float32),
    )(accp, h1, dis2, bc0, wc1)


def _final_body(acc2p_ref, g2_ref, dis_ref, mlp_ref, bc1_ref, out_ref):
    a = acc2p_ref[0] + acc2p_ref[1]
    disb = dis_ref[...]
    xc2 = jnp.maximum(disb * (a + g2_ref[...]) + bc1_ref[...], 0.0)
    out_ref[...] = xc2 + mlp_ref[...]


def _final(acc2p3, g2, dis2, mlp, bc1):
    return pl.pallas_call(
        _final_body,
        grid=(N_PAD // BLK,),
        in_specs=[
            pl.BlockSpec((NC, BLK, 1), lambda i: (0, i, 0)),
            pl.BlockSpec((BLK, 1), lambda i: (i, 0)),
            pl.BlockSpec((BLK, 1), lambda i: (i, 0)),
            pl.BlockSpec((BLK, 1), lambda i: (i, 0)),
            pl.BlockSpec((1, 1), lambda i: (0, 0)),
        ],
        out_specs=pl.BlockSpec((BLK, 1), lambda i: (i, 0)),
        out_shape=jax.ShapeDtypeStruct((N_PAD, 1), jnp.float32),
    )(acc2p3, g2, dis2, mlp, bc1)


def _sc_deg_body(dst_hbm, ew_hbm, out_hbm, acc_sh, zbuf, dstv, ewv, sems):
    c = lax.axis_index("c")
    s = lax.axis_index("s")
    wid = c * NS + s

    def zero_body(i, carry):
        zbuf[pl.ds(i * 16, 16)] = _Z16()
        return carry

    lax.fori_loop(0, RPT // 16, zero_body, None)
    pltpu.sync_copy(zbuf, acc_sh.at[pl.ds(s * RPT, RPT)])
    plsc.subcore_barrier()

    pltpu.sync_copy(dst_hbm.at[wid], dstv)
    pltpu.sync_copy(ew_hbm.at[wid], ewv)

    def fire_body(b, carry):
        pltpu.async_copy(ewv.at[b], acc_sh.at[dstv.at[b]], sems, add=True)
        return carry

    lax.fori_loop(0, NB, fire_body, None)

    def drain_body(b, carry):
        pltpu.make_async_copy(ewv.at[b], acc_sh.at[dstv.at[b]], sems).wait()
        return carry

    lax.fori_loop(0, NB, drain_body, None)
    plsc.subcore_barrier()
    pltpu.sync_copy(acc_sh.at[pl.ds(s * RPT, RPT)],
                    out_hbm.at[c, pl.ds(s * RPT, RPT)])


def _sc_deg(dst3, ew3):
    f = pl.kernel(
        _sc_deg_body,
        out_type=jax.ShapeDtypeStruct((NC, N_PAD), jnp.float32),
        mesh=_mesh(),
        scratch_types=[
            pltpu.VMEM_SHARED((N_PAD,), jnp.float32),
            pltpu.VMEM((RPT,), jnp.float32),
            pltpu.VMEM((NB, EB), jnp.int32),
            pltpu.VMEM((NB, EB), jnp.float32),
            pltpu.SemaphoreType.DMA,
        ],
    )
    return f(dst3, ew3)


def _scale_rows(buf, ewv, b):
    def group_body(gi, carry):
        sl = pl.ds(gi * 16, 16)
        w16 = ewv[b, sl]
        for j in range(16):
            wj = _splat(w16, j)
            e = gi * 16 + j
            for q in range(D_IN // 16):
                qs = pl.ds(q * 16, 16)
                buf[e, qs] = buf[e, qs] * wj
        return carry

    lax.fori_loop(0, EB2 // 16, group_body, None)


def _sc_l1_body(g1_hbm, src_hbm, dst_hbm, ew_hbm, out_hbm,
                acc_sh, srcv, dstv, ewv, buf0, buf1, sem0, sem1):
    c = lax.axis_index("c")
    s = lax.axis_index("s")
    wid = c * NS + s

    def zero_body(i, carry):
        for q in range(D_IN // 16):
            buf0[i, pl.ds(q * 16, 16)] = _Z16()
        return carry

    lax.fori_loop(0, EB2, zero_body, None)
    for kk in range(RPT // EB2):
        pltpu.sync_copy(buf0, acc_sh.at[pl.ds(s * RPT + kk * EB2, EB2)])
    plsc.subcore_barrier()

    dummy = g1_hbm.at[pl.ds(0, EB2)]

    def super_body(sb, carry):
        pltpu.sync_copy(src_hbm.at[wid, sb], srcv)
        pltpu.sync_copy(dst_hbm.at[wid, sb], dstv)
        pltpu.sync_copy(ew_hbm.at[wid, sb], ewv)
        pltpu.async_copy(g1_hbm.at[srcv.at[0]], buf0, sem0)

        def pair_body(k, carry1):
            b0 = 2 * k
            b1 = b0 + 1
            pltpu.async_copy(g1_hbm.at[srcv.at[b1]], buf1, sem1)
            pltpu.make_async_copy(dummy, buf0, sem0).wait()
            _scale_rows(buf0, ewv, b0)
            pltpu.sync_copy(buf0, acc_sh.at[dstv.at[b0]], add=True)

            @pl.when(k < BPS // 2 - 1)
            def _():
                pltpu.async_copy(g1_hbm.at[srcv.at[b0 + 2]], buf0, sem0)

            pltpu.make_async_copy(dummy, buf1, sem1).wait()
            _scale_rows(buf1, ewv, b1)
            pltpu.sync_copy(buf1, acc_sh.at[dstv.at[b1]], add=True)
            return carry1

        lax.fori_loop(0, BPS // 2, pair_body, None)
        return carry

    lax.fori_loop(0, SB, super_body, None)
    plsc.subcore_barrier()
    pltpu.sync_copy(acc_sh.at[pl.ds(s * RPT, RPT)],
                    out_hbm.at[c, pl.ds(s * RPT, RPT)])


def _sc_l1(g1, src4, dst4, ew4):
    f = pl.kernel(
        _sc_l1_body,
        out_type=jax.ShapeDtypeStruct((NC, N_PAD, D_IN), jnp.float32),
        mesh=_mesh(),
        scratch_types=[
            pltpu.VMEM_SHARED((N_PAD, D_IN), jnp.float32),
            pltpu.VMEM((BPS, EB2), jnp.int32),
            pltpu.VMEM((BPS, EB2), jnp.int32),
            pltpu.VMEM((BPS, EB2), jnp.float32),
            pltpu.VMEM((EB2, D_IN), jnp.float32),
            pltpu.VMEM((EB2, D_IN), jnp.float32),
            pltpu.SemaphoreType.DMA,
            pltpu.SemaphoreType.DMA,
        ],
    )
    return f(g1, src4, dst4, ew4)


def _sc_l2_body(g2_hbm, src_hbm, dst_hbm, ew_hbm, out_hbm,
                acc_sh, zbuf, srcv, dstv, ewv, gvals, valv, semg, sems):
    c = lax.axis_index("c")
    s = lax.axis_index("s")
    wid = c * NS + s

    def zero_body(i, carry):
        zbuf[pl.ds(i * 16, 16)] = _Z16()
        return carry

    lax.fori_loop(0, RPT // 16, zero_body, None)
    pltpu.sync_copy(zbuf, acc_sh.at[pl.ds(s * RPT, RPT)])
    plsc.subcore_barrier()

    pltpu.sync_copy(src_hbm.at[wid], srcv)
    pltpu.sync_copy(dst_hbm.at[wid], dstv)
    pltpu.sync_copy(ew_hbm.at[wid], ewv)

    def fire_g(b, carry):
        pltpu.async_copy(g2_hbm.at[srcv.at[b]], gvals.at[b], semg)
        return carry

    lax.fori_loop(0, NB, fire_g, None)

    def drain_g(b, carry):
        pltpu.make_async_copy(g2_hbm.at[srcv.at[b]], gvals.at[b], semg).wait()
        return carry

    lax.fori_loop(0, NB, drain_g, None)

    def scale_b(b, carry):
        for q in range(EB // 16):
            sl = pl.ds(q * 16, 16)
            valv[b, sl] = gvals[b, sl] * ewv[b, sl]
        return carry

    lax.fori_loop(0, NB, scale_b, None)

    def fire_s(b, carry):
        pltpu.async_copy(valv.at[b], acc_sh.at[dstv.at[b]], sems, add=True)
        return carry

    lax.fori_loop(0, NB, fire_s, None)

    def drain_s(b, carry):
        pltpu.make_async_copy(valv.at[b], acc_sh.at[dstv.at[b]], sems).wait()
        return carry

    lax.fori_loop(0, NB, drain_s, None)
    plsc.subcore_barrier()
    pltpu.sync_copy(acc_sh.at[pl.ds(s * RPT, RPT)],
                    out_hbm.at[c, pl.ds(s * RPT, RPT)])


def _sc_l2(g2, src3, dst3, ew3):
    f = pl.kernel(
        _sc_l2_body,
        out_type=jax.ShapeDtypeStruct((NC, N_PAD), jnp.float32),
        mesh=_mesh(),
        scratch_types=[
            pltpu.VMEM_SHARED((N_PAD,), jnp.float32),
            pltpu.VMEM((RPT,), jnp.float32),
            pltpu.VMEM((NB, EB), jnp.int32),
            pltpu.VMEM((NB, EB), jnp.int32),
            pltpu.VMEM((NB, EB), jnp.float32),
            pltpu.VMEM((NB, EB), jnp.float32),
            pltpu.VMEM((NB, EB), jnp.float32),
            pltpu.SemaphoreType.DMA,
            pltpu.SemaphoreType.DMA,
        ],
    )
    return f(g2, src3, dst3, ew3)


def kernel(x, edge_index, edge_feature, W_lin, b_lin, W_c0, b_c0, W_c1, b_c1,
           W_m0, b_m0, W_m1, b_m1):
    src_w = edge_index[0].reshape(NW, EPW)
    dst_w = edge_index[1].reshape(NW, EPW)

    m = jnp.kron(jnp.eye(8, dtype=jnp.float32),
                 jnp.full((16, 1), 1.0 / 16, jnp.float32))
    ew_w = _edge_mean(edge_feature.reshape(E // 8, 128), m).reshape(NW, EPW)

    def _padded(a, pad_to, fill):
        fills = jnp.full((NW, pad_to - EPW), fill, a.dtype)
        return jnp.concatenate([a, fills], axis=1)

    src3 = _padded(src_w, PADW, N).reshape(NW, NB, EB)
    dst3 = _padded(dst_w, PADW, N).reshape(NW, NB, EB)
    ew3 = _padded(ew_w, PADW, 0).reshape(NW, NB, EB)
    src4 = _padded(src_w, PADW2, N).reshape(NW, SB, BPS, EB2)
    dst4 = _padded(dst_w, PADW2, N).reshape(NW, SB, BPS, EB2)
    ew4 = _padded(ew_w, PADW2, 0).reshape(NW, SB, BPS, EB2)

    w_mlp = (W_m1 @ W_m0)[0]
    b_mlp = (b_m1 + W_m1 @ b_m0)[0]

    degp = _sc_deg(dst3, ew3)
    dis = lax.rsqrt(degp[0] + degp[1] + 1.0)
    dis2 = dis[:, None]

    x_pad = jnp.pad(x, ((0, N_PAD - N), (0, 0)))
    g1, mlp = _dense(x_pad, W_lin.T, b_lin[None, :], W_c0.T,
                     w_mlp[None, :], b_mlp[None, None], dis2)

    acc1p = _sc_l1(g1, src4, dst4, ew4)
    g2 = _post1(acc1p, g1, dis2, b_c0[None, :], W_c1)

    acc2p = _sc_l2(g2.reshape(N_PAD), src3, dst3, ew3)

    out = _final(acc2p.reshape(NC, N_PAD, 1), g2, dis2, mlp,
                 b_c1[None, :])
    return out[:N, 0]
